# baseline (speedup 1.0000x reference)
import jax
import jax.numpy as jnp
from jax.experimental import pallas as pl

_K = 4
_EPS = 1e-4


def _blend_block(x_ref, p_ref, shm_ref, o_ref):
    x = x_ref[...]
    p = p_ref[...]
    eye = jax.lax.broadcasted_iota(jnp.int32, (3, 3), 0) == \
        jax.lax.broadcasted_iota(jnp.int32, (3, 3), 1)
    xt = jax.lax.dot_general(
        eye.astype(jnp.float32), x, (((1,), (1,)), ((), ())),
        preferred_element_type=jnp.float32,
        precision=jax.lax.Precision.HIGHEST)
    x2 = ((xt[0:1, :] * xt[0:1, :] + xt[2:3, :] * xt[2:3, :])
          + xt[1:2, :] * xt[1:2, :])
    p2 = ((p[:, 0:1] * p[:, 0:1] + p[:, 2:3] * p[:, 2:3])
          + p[:, 1:2] * p[:, 1:2])
    xp = jax.lax.dot_general(
        p.astype(jnp.bfloat16), xt.astype(jnp.bfloat16),
        (((1,), (0,)), ((), ())),
        preferred_element_type=jnp.float32)
    d2 = x2 + p2 - 2.0 * xp
    m_probes = d2.shape[0]
    iota = jax.lax.broadcasted_iota(jnp.int32, d2.shape, 0)
    work = d2
    wacc = jnp.zeros_like(d2)
    wsum = jnp.zeros((1,) + d2.shape[1:], jnp.float32)
    for _ in range(_K):
        m = jnp.min(work, axis=0, keepdims=True)
        ismin = work <= m
        first = jnp.min(jnp.where(ismin, iota, m_probes), axis=0, keepdims=True)
        onehot = iota == first
        w = 1.0 / (jnp.sqrt(jnp.maximum(m, 0.0)) + _EPS)
        wacc = wacc + jnp.where(onehot, w, 0.0)
        wsum = wsum + w
        work = jnp.where(onehot, jnp.inf, work)
    wn = wacc * (1.0 / wsum)
    o_ref[...] = jax.lax.dot_general(
        wn, shm_ref[...], (((0,), (0,)), ((), ())),
        preferred_element_type=jnp.float32,
        precision=jax.lax.Precision.HIGHEST)


def kernel(xyz, sh_coeffs, probe_positions, active_sh_degree):
    n, _ = xyz.shape
    m, sh_dim, ch = sh_coeffs.shape
    active_dim = (active_sh_degree + 1) ** 2
    mask = (jnp.arange(sh_dim) < active_dim).astype(sh_coeffs.dtype)
    shm = (sh_coeffs * mask[None, :, None]).reshape(m, sh_dim * ch)

    block = 2048
    out = pl.pallas_call(
        _blend_block,
        grid=((n + block - 1) // block,),
        in_specs=[
            pl.BlockSpec((block, 3), lambda i: (i, 0)),
            pl.BlockSpec((m, 3), lambda i: (0, 0)),
            pl.BlockSpec((m, sh_dim * ch), lambda i: (0, 0)),
        ],
        out_specs=pl.BlockSpec((block, sh_dim * ch), lambda i: (i, 0)),
        out_shape=jax.ShapeDtypeStruct((n, sh_dim * ch), jnp.float32),
    )(xyz, probe_positions, shm)
    return out.reshape(n, sh_dim, ch)


# speedup vs baseline: 1.1117x; 1.1117x over previous
import jax
import jax.numpy as jnp
from jax.experimental import pallas as pl

_K = 4
_EPS = 1e-4


def _blend_block(x_ref, x2_ref, p_ref, shm_ref, o_ref):
    x = x_ref[...]
    x2 = x2_ref[...]
    p = p_ref[...]
    p2 = ((p[:, 0:1] * p[:, 0:1] + p[:, 2:3] * p[:, 2:3])
          + p[:, 1:2] * p[:, 1:2])
    xp = jax.lax.dot_general(
        p.astype(jnp.bfloat16), x.astype(jnp.bfloat16),
        (((1,), (1,)), ((), ())),
        preferred_element_type=jnp.float32)
    d2 = x2 + p2 - 2.0 * xp
    m_probes = d2.shape[0]
    iota = jax.lax.broadcasted_iota(jnp.int32, d2.shape, 0)
    work = d2
    wacc = jnp.zeros_like(d2)
    wsum = jnp.zeros((1,) + d2.shape[1:], jnp.float32)
    for _ in range(_K):
        m = jnp.min(work, axis=0, keepdims=True)
        ismin = work <= m
        first = jnp.min(jnp.where(ismin, iota, m_probes), axis=0, keepdims=True)
        onehot = iota == first
        w = 1.0 / (jnp.sqrt(jnp.maximum(m, 0.0)) + _EPS)
        wacc = wacc + jnp.where(onehot, w, 0.0)
        wsum = wsum + w
        work = jnp.where(onehot, jnp.inf, work)
    wn = wacc * (1.0 / wsum)
    o_ref[...] = jax.lax.dot_general(
        wn, shm_ref[...], (((0,), (0,)), ((), ())),
        preferred_element_type=jnp.float32,
        precision=jax.lax.Precision.HIGHEST)


def kernel(xyz, sh_coeffs, probe_positions, active_sh_degree):
    n, _ = xyz.shape
    m, sh_dim, ch = sh_coeffs.shape
    active_dim = (active_sh_degree + 1) ** 2
    mask = (jnp.arange(sh_dim) < active_dim).astype(sh_coeffs.dtype)
    shm = (sh_coeffs * mask[None, :, None]).reshape(m, sh_dim * ch)
    x2 = jnp.sum(xyz * xyz, axis=-1)[None, :]

    block = 2048
    out = pl.pallas_call(
        _blend_block,
        grid=((n + block - 1) // block,),
        in_specs=[
            pl.BlockSpec((block, 3), lambda i: (i, 0)),
            pl.BlockSpec((1, block), lambda i: (0, i)),
            pl.BlockSpec((m, 3), lambda i: (0, 0)),
            pl.BlockSpec((m, sh_dim * ch), lambda i: (0, 0)),
        ],
        out_specs=pl.BlockSpec((block, sh_dim * ch), lambda i: (i, 0)),
        out_shape=jax.ShapeDtypeStruct((n, sh_dim * ch), jnp.float32),
    )(xyz, x2, probe_positions, shm)
    return out.reshape(n, sh_dim, ch)


# speedup vs baseline: 1.3008x; 1.1701x over previous
import jax
import jax.numpy as jnp
from jax.experimental import pallas as pl

_K = 4
_EPS = 1e-4


def _blend_block(xt_ref, p_ref, shm_ref, o_ref):
    xt = xt_ref[...]
    p = p_ref[...]
    x2 = ((xt[0:1, :] * xt[0:1, :] + xt[2:3, :] * xt[2:3, :])
          + xt[1:2, :] * xt[1:2, :])
    p2 = ((p[:, 0:1] * p[:, 0:1] + p[:, 2:3] * p[:, 2:3])
          + p[:, 1:2] * p[:, 1:2])
    xp = jax.lax.dot_general(
        p.astype(jnp.bfloat16), xt.astype(jnp.bfloat16),
        (((1,), (0,)), ((), ())),
        preferred_element_type=jnp.float32)
    d2 = x2 + p2 - 2.0 * xp
    m_probes = d2.shape[0]
    iota = jax.lax.broadcasted_iota(jnp.int32, d2.shape, 0)
    work = d2
    wacc = jnp.zeros_like(d2)
    wsum = jnp.zeros((1,) + d2.shape[1:], jnp.float32)
    for _ in range(_K):
        m = jnp.min(work, axis=0, keepdims=True)
        ismin = work <= m
        first = jnp.min(jnp.where(ismin, iota, m_probes), axis=0, keepdims=True)
        onehot = iota == first
        w = 1.0 / (jnp.sqrt(jnp.maximum(m, 0.0)) + _EPS)
        wacc = wacc + jnp.where(onehot, w, 0.0)
        wsum = wsum + w
        work = jnp.where(onehot, jnp.inf, work)
    wn = wacc * (1.0 / wsum)
    o_ref[...] = jax.lax.dot_general(
        wn, shm_ref[...], (((0,), (0,)), ((), ())),
        preferred_element_type=jnp.float32,
        precision=jax.lax.Precision.HIGHEST)


def kernel(xyz, sh_coeffs, probe_positions, active_sh_degree):
    n, _ = xyz.shape
    m, sh_dim, ch = sh_coeffs.shape
    active_dim = (active_sh_degree + 1) ** 2
    mask = (jnp.arange(sh_dim) < active_dim).astype(sh_coeffs.dtype)
    shm = (sh_coeffs * mask[None, :, None]).reshape(m, sh_dim * ch)
    xt = xyz.T

    block = 4096
    out = pl.pallas_call(
        _blend_block,
        grid=((n + block - 1) // block,),
        in_specs=[
            pl.BlockSpec((3, block), lambda i: (0, i)),
            pl.BlockSpec((m, 3), lambda i: (0, 0)),
            pl.BlockSpec((m, sh_dim * ch), lambda i: (0, 0)),
        ],
        out_specs=pl.BlockSpec((block, sh_dim * ch), lambda i: (i, 0)),
        out_shape=jax.ShapeDtypeStruct((n, sh_dim * ch), jnp.float32),
    )(xt, probe_positions, shm)
    return out.reshape(n, sh_dim, ch)
